# Initial kernel scaffold; baseline (speedup 1.0000x reference)
import jax
import jax.numpy as jnp
from jax.experimental import pallas as pl


def kernel(xyz, features, W1, b1, W2, b2, W3, b3):
    raise NotImplementedError("write your pallas kernel here")


# speedup vs baseline: 4.8003x; 4.8003x over previous
import functools

import jax
import jax.numpy as jnp
from jax import lax
from jax.experimental import pallas as pl
from jax.experimental.pallas import tpu as pltpu
from jax.experimental.pallas import tpu_sc as plsc

_B, _N, _CIN = 8, 4096, 64
_S = _N // 4
_K = 32
_R2 = 0.2 ** 2
_NR = _N // 128
_CPW = (_B * _S) // 32
_TC = 256


def _fps_body(xyzp_ref, npl_ref):
    xp = xyzp_ref[0, 0]
    yp = xyzp_ref[0, 1]
    zp = xyzp_ref[0, 2]
    lin = (lax.broadcasted_iota(jnp.int32, (_NR, 128), 0) * 128
           + lax.broadcasted_iota(jnp.int32, (_NR, 128), 1))
    zero = jnp.float32(0.0)
    sel0 = lin == 0
    cx0 = jnp.sum(jnp.where(sel0, xp, zero))
    cy0 = jnp.sum(jnp.where(sel0, yp, zero))
    cz0 = jnp.sum(jnp.where(sel0, zp, zero))

    def body(i, carry):
        dists, cx, cy, cz = carry
        npl_ref[0, 0, i] = cx
        npl_ref[0, 1, i] = cy
        npl_ref[0, 2, i] = cz
        dx = xp - cx
        dy = yp - cy
        dz = zp - cz
        d = (dx * dx + dy * dy) + dz * dz
        nd = jnp.minimum(dists, d)
        m = jnp.max(nd)
        fi = jnp.min(jnp.where(nd == m, lin, jnp.int32(1 << 30)))
        selm = lin == fi
        ncx = jnp.sum(jnp.where(selm, xp, zero))
        ncy = jnp.sum(jnp.where(selm, yp, zero))
        ncz = jnp.sum(jnp.where(selm, zp, zero))
        return nd, ncx, ncy, ncz

    dists0 = jnp.full((_NR, 128), 1e10, jnp.float32)
    lax.fori_loop(0, _S, body, (dists0, cx0, cy0, cz0))


def _fps(xyzp4):
    return pl.pallas_call(
        _fps_body,
        grid=(_B,),
        in_specs=[pl.BlockSpec((1, 3, _NR, 128), lambda b: (b, 0, 0, 0))],
        out_specs=pl.BlockSpec((1, 3, _S), lambda b: (b, 0, 0),
                               memory_space=pltpu.SMEM),
        out_shape=jax.ShapeDtypeStruct((_B, 3, _S), jnp.float32),
    )(xyzp4)


def _prep_body(xyz8_ref, ft_ref, nx8_ref, w1x_ref, w1f_ref, b1_ref,
               t_ref, cc_ref):
    w1x = w1x_ref[...]
    tv = (jnp.dot(xyz8_ref[0], w1x, preferred_element_type=jnp.float32)
          + jnp.dot(ft_ref[0], w1f_ref[...],
                    preferred_element_type=jnp.float32)
          + b1_ref[...])
    t_ref[0] = jnp.concatenate(
        [tv, jnp.zeros((_N, 64), jnp.float32)], axis=1)
    cc_ref[0] = jnp.dot(nx8_ref[0], w1x, preferred_element_type=jnp.float32)


def _prep(xyz8, featsT, nxyz8, w1x8, w1f, b1r):
    return pl.pallas_call(
        _prep_body,
        grid=(_B,),
        in_specs=[
            pl.BlockSpec((1, _N, 8), lambda b: (b, 0, 0)),
            pl.BlockSpec((1, _N, _CIN), lambda b: (b, 0, 0)),
            pl.BlockSpec((1, _S, 8), lambda b: (b, 0, 0)),
            pl.BlockSpec((8, 64), lambda b: (0, 0)),
            pl.BlockSpec((_CIN, 64), lambda b: (0, 0)),
            pl.BlockSpec((1, 64), lambda b: (0, 0)),
        ],
        out_specs=[
            pl.BlockSpec((1, _N, 128), lambda b: (b, 0, 0)),
            pl.BlockSpec((1, _S, 64), lambda b: (b, 0, 0)),
        ],
        out_shape=[
            jax.ShapeDtypeStruct((_B, _N, 128), jnp.float32),
            jax.ShapeDtypeStruct((_B, _S, 64), jnp.float32),
        ],
    )(xyz8, featsT, nxyz8, w1x8, w1f, b1r)


def _ballq_body(xyzpl, nxyzpl, t2, g_out,
                x_v, y_v, z_v, nx_v, ny_v, nz_v, idxT, idxbuf, rows_v, sem):
    wid = lax.axis_index("s") * 2 + lax.axis_index("c")
    b = wid // 4
    q = wid % 4
    c0 = q * _CPW

    pltpu.sync_copy(xyzpl.at[b * 3 + 0], x_v)
    pltpu.sync_copy(xyzpl.at[b * 3 + 1], y_v)
    pltpu.sync_copy(xyzpl.at[b * 3 + 2], z_v)
    pltpu.sync_copy(nxyzpl.at[b * 3 + 0, pl.ds(c0, _CPW)],
                    nx_v.at[pl.ds(0, _CPW)])
    pltpu.sync_copy(nxyzpl.at[b * 3 + 1, pl.ds(c0, _CPW)],
                    ny_v.at[pl.ds(0, _CPW)])
    pltpu.sync_copy(nxyzpl.at[b * 3 + 2, pl.ds(c0, _CPW)],
                    nz_v.at[pl.ds(0, _CPW)])

    lane = lax.iota(jnp.int32, 16)
    gbase = b * _N

    def per_center(cl, _):
        cx = nx_v[pl.ds(cl, 16)][0]
        cy = ny_v[pl.ds(cl, 16)][0]
        cz = nz_v[pl.ds(cl, 16)][0]

        def super_chunk(t, cnt):
            def scan_chunk(c2):
                for u in range(16):
                    base = t * 256 + u * 16
                    xv = x_v[pl.ds(base, 16)]
                    yv = y_v[pl.ds(base, 16)]
                    zv = z_v[pl.ds(base, 16)]
                    dx = xv - cx
                    dy = yv - cy
                    dz = zv - cz
                    d2 = (dx * dx + dy * dy) + dz * dz
                    m = d2 <= _R2
                    plsc.store_compressed(idxbuf.at[pl.ds(c2, 16)],
                                          lane + (gbase + base), mask=m)
                    c2 = c2 + plsc.all_reduce_population_count(m)[0]
                return c2

            return lax.cond(cnt < _K, scan_chunk, lambda c2: c2, cnt)

        cnt = lax.fori_loop(0, _N // 256, super_chunk, jnp.int32(0))

        v1 = idxbuf[pl.ds(0, 16)]
        v2 = idxbuf[pl.ds(16, 16)]
        first = v1[0]
        vf1 = jnp.where(lane < cnt, v1, first)
        vf2 = jnp.where(lane + 16 < cnt, v2, first)
        plsc.store_scatter(idxT, [lane * _CPW + cl], vf1)
        plsc.store_scatter(idxT, [(lane + 16) * _CPW + cl], vf2)
        return 0

    lax.fori_loop(0, _CPW, per_center, 0)

    row0 = (b * _K) * _S + c0

    def emit(j, _):
        pltpu.async_copy(t2.at[idxT.at[pl.ds(j * _CPW, _CPW)]],
                         rows_v, sem).wait()
        pltpu.sync_copy(rows_v, g_out.at[pl.ds(row0 + j * _S, _CPW)])
        return 0

    lax.fori_loop(0, _K, emit, 0)


def _ballq(xyzpl, nxyzpl, t2):
    mesh = plsc.VectorSubcoreMesh(core_axis_name="c", subcore_axis_name="s")
    kfn = pl.kernel(
        _ballq_body,
        out_type=jax.ShapeDtypeStruct((_B * _K * _S, 128), jnp.float32),
        mesh=mesh,
        compiler_params=pltpu.CompilerParams(needs_layout_passes=False),
        scratch_types=[
            pltpu.VMEM((_N,), jnp.float32),
            pltpu.VMEM((_N,), jnp.float32),
            pltpu.VMEM((_N,), jnp.float32),
            pltpu.VMEM((_CPW + 16,), jnp.float32),
            pltpu.VMEM((_CPW + 16,), jnp.float32),
            pltpu.VMEM((_CPW + 16,), jnp.float32),
            pltpu.VMEM((_K * _CPW,), jnp.int32),
            pltpu.VMEM((320,), jnp.int32),
            pltpu.VMEM((_CPW, 128), jnp.float32),
            pltpu.SemaphoreType.DMA,
        ],
    )
    return kfn(xyzpl, nxyzpl, t2)


def _mlp_body(g_ref, cc_ref, w2_ref, b2_ref, w3_ref, b3_ref, out_ref):
    cc = cc_ref[0]
    w2 = w2_ref[...]
    b2 = b2_ref[...]
    w3 = w3_ref[...]
    b3 = b3_ref[...]

    def body(j, acc):
        h1 = jnp.maximum(g_ref[0, j][:, 0:64] - cc, 0.0)
        h2 = jnp.maximum(
            jnp.dot(h1, w2, preferred_element_type=jnp.float32) + b2, 0.0)
        h3 = jnp.maximum(
            jnp.dot(h2, w3, preferred_element_type=jnp.float32) + b3, 0.0)
        return jnp.maximum(acc, h3)

    out_ref[0] = lax.fori_loop(
        0, _K, body, jnp.full((_TC, 128), -jnp.inf, jnp.float32))


def _mlp(g4, cc, w2, b2r, w3, b3r):
    return pl.pallas_call(
        _mlp_body,
        grid=(_B, _S // _TC),
        in_specs=[
            pl.BlockSpec((1, _K, _TC, 128), lambda b, t: (b, 0, t, 0)),
            pl.BlockSpec((1, _TC, 64), lambda b, t: (b, t, 0)),
            pl.BlockSpec((64, 64), lambda b, t: (0, 0)),
            pl.BlockSpec((1, 64), lambda b, t: (0, 0)),
            pl.BlockSpec((64, 128), lambda b, t: (0, 0)),
            pl.BlockSpec((1, 128), lambda b, t: (0, 0)),
        ],
        out_specs=pl.BlockSpec((1, _TC, 128), lambda b, t: (b, t, 0)),
        out_shape=jax.ShapeDtypeStruct((_B, _S, 128), jnp.float32),
    )(g4, cc, w2, b2r, w3, b3r)


def kernel(xyz, features, W1, b1, W2, b2, W3, b3):
    xyzp = jnp.transpose(xyz, (0, 2, 1))
    npl = _fps(xyzp.reshape(_B, 3, _NR, 128))
    new_xyz = jnp.transpose(npl, (0, 2, 1))

    zeros5n = jnp.zeros((_B, _N, 5), jnp.float32)
    zeros5s = jnp.zeros((_B, _S, 5), jnp.float32)
    xyz8 = jnp.concatenate([xyz, zeros5n], axis=-1)
    nxyz8 = jnp.concatenate([new_xyz, zeros5s], axis=-1)
    featsT = jnp.transpose(features, (0, 2, 1))
    w1x8 = jnp.concatenate([W1[:3], jnp.zeros((5, 64), jnp.float32)], axis=0)

    t_tab, cc = _prep(xyz8, featsT, nxyz8, w1x8, W1[3:], b1.reshape(1, 64))

    g = _ballq(xyzp.reshape(_B * 3, _N), npl.reshape(_B * 3, _S),
               t_tab.reshape(_B * _N, 128))

    out = _mlp(g.reshape(_B, _K, _S, 128), cc, W2, b2.reshape(1, 64),
               W3, b3.reshape(1, 128))
    return new_xyz, jnp.transpose(out, (0, 2, 1))


# speedup vs baseline: 7.1458x; 1.4886x over previous
import functools

import jax
import jax.numpy as jnp
from jax import lax
from jax.experimental import pallas as pl
from jax.experimental.pallas import tpu as pltpu
from jax.experimental.pallas import tpu_sc as plsc

_B, _N, _CIN = 8, 4096, 64
_S = _N // 4
_K = 32
_R2 = 0.2 ** 2
_NR = _N // 128
_CPW = (_B * _S) // 32
_TC = 256


def _fps_body(xyzp_ref, xyzs_ref, npl_ref, dists_ref):
    lin = (lax.broadcasted_iota(jnp.int32, (_NR, 128), 0) * 128
           + lax.broadcasted_iota(jnp.int32, (_NR, 128), 1))
    big = jnp.int32(1 << 30)
    for u in range(2):
        dists_ref[u] = jnp.full((_NR, 128), 1e10, jnp.float32)

    def body(i, carry):
        f0, f1 = carry
        fs = []
        for u, f in ((0, f0), (1, f1)):
            cx = xyzs_ref[0, u, 0, f]
            cy = xyzs_ref[0, u, 1, f]
            cz = xyzs_ref[0, u, 2, f]
            npl_ref[0, u, 0, i] = cx
            npl_ref[0, u, 1, i] = cy
            npl_ref[0, u, 2, i] = cz
            dx = xyzp_ref[0, u, 0] - cx
            dy = xyzp_ref[0, u, 1] - cy
            dz = xyzp_ref[0, u, 2] - cz
            d = (dx * dx + dy * dy) + dz * dz
            nd = jnp.minimum(dists_ref[u], d)
            dists_ref[u] = nd
            m = jnp.max(nd)
            fs.append(jnp.min(jnp.where(nd == m, lin, big)))
        return fs[0], fs[1]

    lax.fori_loop(0, _S, body, (jnp.int32(0), jnp.int32(0)))


def _fps(xyzp4, xyzs):
    return pl.pallas_call(
        _fps_body,
        grid=(_B // 2,),
        in_specs=[
            pl.BlockSpec((1, 2, 3, _NR, 128), lambda b: (b, 0, 0, 0, 0)),
            pl.BlockSpec((1, 2, 3, _N), lambda b: (b, 0, 0, 0),
                         memory_space=pltpu.SMEM),
        ],
        out_specs=pl.BlockSpec((1, 2, 3, _S), lambda b: (b, 0, 0, 0),
                               memory_space=pltpu.SMEM),
        out_shape=jax.ShapeDtypeStruct((_B // 2, 2, 3, _S), jnp.float32),
        scratch_shapes=[pltpu.VMEM((2, _NR, 128), jnp.float32)],
    )(xyzp4, xyzs)


def _prep_body(xyz8_ref, ft_ref, nx8_ref, w1x_ref, w1f_ref, b1_ref,
               t_ref, cc_ref):
    w1x = w1x_ref[...]
    tv = (jnp.dot(xyz8_ref[0], w1x, preferred_element_type=jnp.float32)
          + jnp.dot(ft_ref[0], w1f_ref[...],
                    preferred_element_type=jnp.float32)
          + b1_ref[...])
    t_ref[0] = jnp.concatenate(
        [tv, jnp.zeros((_N, 64), jnp.float32)], axis=1)
    cc_ref[0] = jnp.dot(nx8_ref[0], w1x, preferred_element_type=jnp.float32)


def _prep(xyz8, featsT, nxyz8, w1x8, w1f, b1r):
    return pl.pallas_call(
        _prep_body,
        grid=(_B,),
        in_specs=[
            pl.BlockSpec((1, _N, 8), lambda b: (b, 0, 0)),
            pl.BlockSpec((1, _N, _CIN), lambda b: (b, 0, 0)),
            pl.BlockSpec((1, _S, 8), lambda b: (b, 0, 0)),
            pl.BlockSpec((8, 64), lambda b: (0, 0)),
            pl.BlockSpec((_CIN, 64), lambda b: (0, 0)),
            pl.BlockSpec((1, 64), lambda b: (0, 0)),
        ],
        out_specs=[
            pl.BlockSpec((1, _N, 128), lambda b: (b, 0, 0)),
            pl.BlockSpec((1, _S, 64), lambda b: (b, 0, 0)),
        ],
        out_shape=[
            jax.ShapeDtypeStruct((_B, _N, 128), jnp.float32),
            jax.ShapeDtypeStruct((_B, _S, 64), jnp.float32),
        ],
    )(xyz8, featsT, nxyz8, w1x8, w1f, b1r)


def _ballq_body(xyzpl, nxyzpl, t2, g_out,
                x_v, y_v, z_v, nx_v, ny_v, nz_v, idxT, idxbuf, rows_v, sem):
    wid = lax.axis_index("s") * 2 + lax.axis_index("c")
    b = wid // 4
    q = wid % 4
    c0 = q * _CPW

    pltpu.sync_copy(xyzpl.at[b * 3 + 0], x_v)
    pltpu.sync_copy(xyzpl.at[b * 3 + 1], y_v)
    pltpu.sync_copy(xyzpl.at[b * 3 + 2], z_v)
    pltpu.sync_copy(nxyzpl.at[b * 3 + 0, pl.ds(c0, _CPW)],
                    nx_v.at[pl.ds(0, _CPW)])
    pltpu.sync_copy(nxyzpl.at[b * 3 + 1, pl.ds(c0, _CPW)],
                    ny_v.at[pl.ds(0, _CPW)])
    pltpu.sync_copy(nxyzpl.at[b * 3 + 2, pl.ds(c0, _CPW)],
                    nz_v.at[pl.ds(0, _CPW)])

    lane = lax.iota(jnp.int32, 16)
    gbase = b * _N

    def per_center(cl, _):
        cx = nx_v[pl.ds(cl, 16)][0]
        cy = ny_v[pl.ds(cl, 16)][0]
        cz = nz_v[pl.ds(cl, 16)][0]

        def super_chunk(t, cnt):
            def scan_chunk(c2):
                for u in range(16):
                    base = t * 256 + u * 16
                    xv = x_v[pl.ds(base, 16)]
                    yv = y_v[pl.ds(base, 16)]
                    zv = z_v[pl.ds(base, 16)]
                    dx = xv - cx
                    dy = yv - cy
                    dz = zv - cz
                    d2 = (dx * dx + dy * dy) + dz * dz
                    m = d2 <= _R2
                    plsc.store_compressed(idxbuf.at[pl.ds(c2, 16)],
                                          lane + (gbase + base), mask=m)
                    c2 = c2 + plsc.all_reduce_population_count(m)[0]
                return c2

            return lax.cond(cnt < _K, scan_chunk, lambda c2: c2, cnt)

        cnt = lax.fori_loop(0, _N // 256, super_chunk, jnp.int32(0))

        v1 = idxbuf[pl.ds(0, 16)]
        v2 = idxbuf[pl.ds(16, 16)]
        first = v1[0]
        vf1 = jnp.where(lane < cnt, v1, first)
        vf2 = jnp.where(lane + 16 < cnt, v2, first)
        plsc.store_scatter(idxT, [lane * _CPW + cl], vf1)
        plsc.store_scatter(idxT, [(lane + 16) * _CPW + cl], vf2)
        return 0

    lax.fori_loop(0, _CPW, per_center, 0)

    row0 = (b * _K) * _S + c0

    def emit(j, _):
        pltpu.async_copy(t2.at[idxT.at[pl.ds(j * _CPW, _CPW)]],
                         rows_v, sem).wait()
        pltpu.sync_copy(rows_v, g_out.at[pl.ds(row0 + j * _S, _CPW)])
        return 0

    lax.fori_loop(0, _K, emit, 0)


def _ballq(xyzpl, nxyzpl, t2):
    mesh = plsc.VectorSubcoreMesh(core_axis_name="c", subcore_axis_name="s")
    kfn = pl.kernel(
        _ballq_body,
        out_type=jax.ShapeDtypeStruct((_B * _K * _S, 128), jnp.float32),
        mesh=mesh,
        compiler_params=pltpu.CompilerParams(needs_layout_passes=False),
        scratch_types=[
            pltpu.VMEM((_N,), jnp.float32),
            pltpu.VMEM((_N,), jnp.float32),
            pltpu.VMEM((_N,), jnp.float32),
            pltpu.VMEM((_CPW + 16,), jnp.float32),
            pltpu.VMEM((_CPW + 16,), jnp.float32),
            pltpu.VMEM((_CPW + 16,), jnp.float32),
            pltpu.VMEM((_K * _CPW,), jnp.int32),
            pltpu.VMEM((320,), jnp.int32),
            pltpu.VMEM((_CPW, 128), jnp.float32),
            pltpu.SemaphoreType.DMA,
        ],
    )
    return kfn(xyzpl, nxyzpl, t2)


def _mlp_body(g_ref, cc_ref, w2_ref, b2_ref, w3_ref, b3_ref, out_ref):
    cc = cc_ref[0]
    w2 = w2_ref[...]
    b2 = b2_ref[...]
    w3 = w3_ref[...]
    b3 = b3_ref[...]

    def body(j, acc):
        h1 = jnp.maximum(g_ref[0, j][:, 0:64] - cc, 0.0)
        h2 = jnp.maximum(
            jnp.dot(h1, w2, preferred_element_type=jnp.float32) + b2, 0.0)
        h3 = jnp.maximum(
            jnp.dot(h2, w3, preferred_element_type=jnp.float32) + b3, 0.0)
        return jnp.maximum(acc, h3)

    out_ref[0] = lax.fori_loop(
        0, _K, body, jnp.full((_TC, 128), -jnp.inf, jnp.float32))


def _mlp(g4, cc, w2, b2r, w3, b3r):
    return pl.pallas_call(
        _mlp_body,
        grid=(_B, _S // _TC),
        in_specs=[
            pl.BlockSpec((1, _K, _TC, 128), lambda b, t: (b, 0, t, 0)),
            pl.BlockSpec((1, _TC, 64), lambda b, t: (b, t, 0)),
            pl.BlockSpec((64, 64), lambda b, t: (0, 0)),
            pl.BlockSpec((1, 64), lambda b, t: (0, 0)),
            pl.BlockSpec((64, 128), lambda b, t: (0, 0)),
            pl.BlockSpec((1, 128), lambda b, t: (0, 0)),
        ],
        out_specs=pl.BlockSpec((1, _TC, 128), lambda b, t: (b, t, 0)),
        out_shape=jax.ShapeDtypeStruct((_B, _S, 128), jnp.float32),
    )(g4, cc, w2, b2r, w3, b3r)


def kernel(xyz, features, W1, b1, W2, b2, W3, b3):
    xyzp = jnp.transpose(xyz, (0, 2, 1))
    npl = _fps(xyzp.reshape(_B // 2, 2, 3, _NR, 128),
               xyzp.reshape(_B // 2, 2, 3, _N)).reshape(_B, 3, _S)
    new_xyz = jnp.transpose(npl, (0, 2, 1))

    zeros5n = jnp.zeros((_B, _N, 5), jnp.float32)
    zeros5s = jnp.zeros((_B, _S, 5), jnp.float32)
    xyz8 = jnp.concatenate([xyz, zeros5n], axis=-1)
    nxyz8 = jnp.concatenate([new_xyz, zeros5s], axis=-1)
    featsT = jnp.transpose(features, (0, 2, 1))
    w1x8 = jnp.concatenate([W1[:3], jnp.zeros((5, 64), jnp.float32)], axis=0)

    t_tab, cc = _prep(xyz8, featsT, nxyz8, w1x8, W1[3:], b1.reshape(1, 64))

    g = _ballq(xyzp.reshape(_B * 3, _N), npl.reshape(_B * 3, _S),
               t_tab.reshape(_B * _N, 128))

    out = _mlp(g.reshape(_B, _K, _S, 128), cc, W2, b2.reshape(1, 64),
               W3, b3.reshape(1, 128))
    return new_xyz, jnp.transpose(out, (0, 2, 1))
